# baseline (speedup 1.0000x reference)
import jax
import jax.numpy as jnp
from jax.experimental import pallas as pl
from jax.experimental.pallas import tpu as pltpu

_NUM_BLOCKS = 4


def _make_body(chunk_rows):
    def _body(x_vmem, o_hbm, sem):
        i = pl.program_id(0)
        copy = pltpu.make_async_copy(
            x_vmem, o_hbm.at[pl.ds(i * chunk_rows, chunk_rows), :], sem)
        copy.start()
        copy.wait()

    return _body


def kernel(x, idx, label):
    del idx, label
    rows, cols = x.shape
    chunk_rows = rows // _NUM_BLOCKS
    return pl.pallas_call(
        _make_body(chunk_rows),
        out_shape=jax.ShapeDtypeStruct(x.shape, x.dtype),
        grid=(_NUM_BLOCKS,),
        in_specs=[pl.BlockSpec((chunk_rows, cols), lambda i: (i, 0))],
        out_specs=pl.BlockSpec(memory_space=pl.ANY),
        scratch_shapes=[pltpu.SemaphoreType.DMA],
    )(x)
